# Initial kernel scaffold; baseline (speedup 1.0000x reference)
import jax
import jax.numpy as jnp
from jax.experimental import pallas as pl


def kernel(query, reference_points, input_flatten, input_spatial_shapes, input_level_start_index, W_out, b_out):
    raise NotImplementedError("write your pallas kernel here")


# speedup vs baseline: 440.2650x; 440.2650x over previous
import functools

import numpy as np
import jax
import jax.numpy as jnp
from jax import lax
from jax.experimental import pallas as pl
from jax.experimental.pallas import tpu as pltpu
from jax.experimental.pallas import tpu_sc as plsc

DM = 256
NL = 4
NH = 8
NP = 4
NBATCH = 2
LQ = 1024
HH = 32
WW = 32
HW = HH * WW
LIN = NL * HW
DH = DM // NH
MAXG = (HH - 1) * (WW - 1)
DELTAS = (-1 - HH, -HH, 1 - HH, -1, 0, 1, -1 + HH, HH, 1 + HH)

TQ = 256

NGROUP = NBATCH * NH * LQ
NROWS = NBATCH * NL * HW * NH
NW = 32
GPT = NGROUP // NW
CH = 64
NCHUNK = GPT // CH


def _build_stencil() -> np.ndarray:
    t = np.arange(HW)
    a = np.zeros((HW, HW), np.float32)
    for d in DELTAS:
        n = np.clip(t + d, 0, MAXG)
        w = n // HH
        h = n - w * HH
        for dy in (-1, 0):
            for dx in (-1, 0):
                y = h + dy
                x = w + dx
                valid = (y >= 0) & (x >= 0)
                np.add.at(a, (t[valid], (y * WW + x)[valid]), 0.25)
    return a * (1.0 / (NL * NP * 9))


_A_NP = _build_stencil()


def _topk_body(q_ref, v_ref, idx_ref, locw_ref, loch_ref):
    q = q_ref[0, 0]
    v = v_ref[0, 0]
    corr = lax.dot_general(q, v, (((1,), (1,)), ((), ())),
                           preferred_element_type=jnp.float32)
    iota = lax.broadcasted_iota(jnp.int32, (TQ, HW), 1)
    cols = []
    for l in range(NL):
        c = corr[:, l * HW:(l + 1) * HW]
        for _ in range(NP):
            mx = jnp.max(c, axis=1, keepdims=True)
            ip = jnp.min(jnp.where(c == mx, iota, HW), axis=1, keepdims=True)
            cols.append(ip)
            c = jnp.where(iota == ip, -jnp.inf, c)
    idx16 = jnp.concatenate(cols, axis=1)
    idx_ref[0, 0] = idx16
    ws, hs = [], []
    for d in DELTAS:
        n = jnp.clip(idx16 + d, 0, MAXG)
        w = n // HH
        ws.append(w)
        hs.append(n - w * HH)
    scale = jnp.float32(1.0 / WW)
    locw_ref[0, 0] = jnp.concatenate(ws, axis=1).astype(jnp.float32) * scale
    loch_ref[0, 0] = jnp.concatenate(hs, axis=1).astype(jnp.float32) * scale


def _topk_call(qh, vh):
    grid = (NBATCH, NH, LQ // TQ)
    return pl.pallas_call(
        _topk_body,
        grid=grid,
        in_specs=[
            pl.BlockSpec((1, 1, TQ, DH), lambda b, m, i: (b, m, i, 0)),
            pl.BlockSpec((1, 1, LIN, DH), lambda b, m, i: (b, m, 0, 0)),
        ],
        out_specs=[
            pl.BlockSpec((1, 1, TQ, 16), lambda b, m, i: (b, m, i, 0)),
            pl.BlockSpec((1, 1, TQ, 144), lambda b, m, i: (b, m, i, 0)),
            pl.BlockSpec((1, 1, TQ, 144), lambda b, m, i: (b, m, i, 0)),
        ],
        out_shape=[
            jax.ShapeDtypeStruct((NBATCH, NH, LQ, 16), jnp.int32),
            jax.ShapeDtypeStruct((NBATCH, NH, LQ, 144), jnp.float32),
            jax.ShapeDtypeStruct((NBATCH, NH, LQ, 144), jnp.float32),
        ],
    )(qh, vh)


def _table_body(a_ref, v_ref, t_ref):
    t_ref[0, 0] = jnp.dot(a_ref[...], v_ref[0, 0],
                          preferred_element_type=jnp.float32)


def _table_call(a, vl):
    return pl.pallas_call(
        _table_body,
        grid=(NBATCH, NL),
        in_specs=[
            pl.BlockSpec((HW, HW), lambda b, l: (0, 0)),
            pl.BlockSpec((1, 1, HW, DM), lambda b, l: (b, l, 0, 0)),
        ],
        out_specs=pl.BlockSpec((1, 1, HW, DM), lambda b, l: (b, l, 0, 0)),
        out_shape=jax.ShapeDtypeStruct((NBATCH, NL, HW, DM), jnp.float32),
    )(a, vl)


def _proj_body(x_ref, w_ref, b_ref, o_ref):
    o_ref[...] = lax.dot_general(
        x_ref[...], w_ref[...], (((1,), (1,)), ((), ())),
        preferred_element_type=jnp.float32) + b_ref[...]


def _proj_call(x, w, b):
    n = x.shape[0]
    return pl.pallas_call(
        _proj_body,
        grid=(1,),
        in_specs=[
            pl.BlockSpec((n, DM), lambda i: (0, 0)),
            pl.BlockSpec((DM, DM), lambda i: (0, 0)),
            pl.BlockSpec((1, DM), lambda i: (0, 0)),
        ],
        out_specs=pl.BlockSpec((n, DM), lambda i: (0, 0)),
        out_shape=jax.ShapeDtypeStruct((n, DM), jnp.float32),
    )(x, w, b)


def _sc_gather(tbl2d, idx2d):
    mesh = plsc.VectorSubcoreMesh(core_axis_name="c", subcore_axis_name="s")

    @functools.partial(
        pl.kernel,
        mesh=mesh,
        compiler_params=pltpu.CompilerParams(use_tc_tiling_on_sc=False),
        out_type=jax.ShapeDtypeStruct((NGROUP, DH), jnp.float32),
        scratch_types=[
            pltpu.VMEM((CH, 16), jnp.int32),
            pltpu.VMEM((CH * 16 // 128, 128), jnp.int32),
            pltpu.VMEM((CH * 16, DH), jnp.float32),
            pltpu.VMEM((CH, DH), jnp.float32),
            pltpu.SemaphoreType.DMA,
        ],
    )
    def k(tbl_hbm, idx_hbm, out_hbm, idx_v, gidx_v, rows_v, out_v, sem):
        wid = lax.axis_index("s") * 2 + lax.axis_index("c")
        g0 = wid * GPT
        b = g0 >> 13
        m = (g0 >> 10) & 7
        lane = lax.iota(jnp.int32, 16)
        add = ((lane >> 2) << 13) + ((b << 15) + m)

        def chunk(ci, carry):
            base = g0 + ci * CH
            pltpu.sync_copy(idx_hbm.at[pl.ds(base, CH)], idx_v)
            for i in range(CH):
                gv = (idx_v[i, :] << 3) + add
                gidx_v[i // 8, pl.ds((i % 8) * 16, 16)] = gv
            handles = [
                pltpu.async_copy(tbl_hbm.at[gidx_v.at[j]],
                                 rows_v.at[pl.ds(j * 128, 128)], sem)
                for j in range(CH * 16 // 128)
            ]
            for h in handles:
                h.wait()

            def red(g, c2):
                r0 = g * 16
                acc0 = rows_v[r0, pl.ds(0, 16)]
                acc1 = rows_v[r0, pl.ds(16, 16)]
                for j in range(1, 16):
                    acc0 = acc0 + rows_v[r0 + j, pl.ds(0, 16)]
                    acc1 = acc1 + rows_v[r0 + j, pl.ds(16, 16)]
                out_v[g, pl.ds(0, 16)] = acc0
                out_v[g, pl.ds(16, 16)] = acc1
                return c2

            lax.fori_loop(0, CH, red, 0)
            pltpu.sync_copy(out_v, out_hbm.at[pl.ds(base, CH)])
            return carry

        lax.fori_loop(0, NCHUNK, chunk, 0)

    return k(tbl2d, idx2d)


def kernel(query, reference_points, input_flatten, input_spatial_shapes,
           input_level_start_index, W_out, b_out):
    qh = query.reshape(NBATCH, LQ, NH, DH).transpose(0, 2, 1, 3)
    vh = input_flatten.reshape(NBATCH, LIN, NH, DH).transpose(0, 2, 1, 3)
    idx, locw, loch = _topk_call(qh, vh)

    a = jnp.asarray(_A_NP)
    tbl = _table_call(a, input_flatten.reshape(NBATCH, NL, HW, DM))

    pre = _sc_gather(tbl.reshape(NROWS, DH), idx.reshape(NGROUP, 16))
    pre = pre.reshape(NBATCH, NH, LQ, DH).transpose(0, 2, 1, 3)
    pre = pre.reshape(NBATCH * LQ, DM)

    out = _proj_call(pre, W_out, b_out.reshape(1, DM))
    out = out.reshape(NBATCH, LQ, DM)

    def arrange(t):
        t = t.reshape(NBATCH, NH, LQ, 9, NL, NP)
        t = t.transpose(0, 2, 1, 4, 3, 5)
        return t.reshape(NBATCH, LQ, NH, NL, 9 * NP)

    loc = jnp.stack([arrange(locw), arrange(loch)], axis=-1)
    return out, loc
